# baseline (speedup 1.0000x reference)
import jax
import jax.numpy as jnp
from jax.experimental import pallas as pl
from jax.experimental.pallas import tpu as pltpu


def _passthrough_copy(t_in, id_in, t_out, id_out, t_sem, id_sem):
    t_copy = pltpu.make_async_copy(t_in, t_out, t_sem)
    id_copy = pltpu.make_async_copy(id_in, id_out, id_sem)
    t_copy.start()
    id_copy.start()
    t_copy.wait()
    id_copy.wait()


def kernel(t, id=None):
    if id is None:
        id = jnp.tile(jnp.array([[0]], dtype=jnp.int32), (t.shape[0], 1))
    return pl.pallas_call(
        _passthrough_copy,
        out_shape=(
            jax.ShapeDtypeStruct(t.shape, t.dtype),
            jax.ShapeDtypeStruct(id.shape, id.dtype),
        ),
        in_specs=[
            pl.BlockSpec(memory_space=pl.ANY),
            pl.BlockSpec(memory_space=pl.ANY),
        ],
        out_specs=(
            pl.BlockSpec(memory_space=pl.ANY),
            pl.BlockSpec(memory_space=pl.ANY),
        ),
        scratch_shapes=[pltpu.SemaphoreType.DMA, pltpu.SemaphoreType.DMA],
    )(t, id)


# speedup vs baseline: 13.7587x; 13.7587x over previous
import jax
import jax.numpy as jnp
from jax.experimental import pallas as pl
from jax.experimental.pallas import tpu as pltpu

_GRID = 8


def _copy_block(t_in, id_in, t_out, id_out):
    t_out[...] = t_in[...]
    id_out[...] = id_in[...]


def kernel(t, id=None):
    if id is None:
        id = jnp.tile(jnp.array([[0]], dtype=jnp.int32), (t.shape[0], 1))
    rows = t.shape[0]
    blk = rows // _GRID
    return pl.pallas_call(
        _copy_block,
        grid=(_GRID,),
        out_shape=(
            jax.ShapeDtypeStruct(t.shape, t.dtype),
            jax.ShapeDtypeStruct(id.shape, id.dtype),
        ),
        in_specs=[
            pl.BlockSpec((blk, t.shape[1]), lambda i: (i, 0)),
            pl.BlockSpec((blk, 1), lambda i: (i, 0)),
        ],
        out_specs=(
            pl.BlockSpec((blk, t.shape[1]), lambda i: (i, 0)),
            pl.BlockSpec((blk, 1), lambda i: (i, 0)),
        ),
        compiler_params=pltpu.CompilerParams(
            dimension_semantics=("arbitrary",),
        ),
    )(t, id)
